# Initial kernel scaffold; baseline (speedup 1.0000x reference)
import jax
import jax.numpy as jnp
from jax.experimental import pallas as pl


def kernel(actors, nodes, actor_ctrs, graph_ctrs, graph_turn, graph_control, graph_intersect, pre_u, pre_v, suc_u, suc_v, left_u, left_v, right_u, right_v, params):
    raise NotImplementedError("write your pallas kernel here")


# speedup vs baseline: 4.8449x; 4.8449x over previous
import functools

import jax
import jax.numpy as jnp
from jax import lax
from jax.experimental import pallas as pl
from jax.experimental.pallas import tpu as pltpu
from jax.experimental.pallas import tpu_sc as plsc

D = 128
NA = 64
BN = 256
CB = 128
EB = 128
NWORK = 32


def _ln(x, g, b):
    m = jnp.mean(x, axis=-1, keepdims=True)
    d = x - m
    v = jnp.mean(d * d, axis=-1, keepdims=True)
    return d * lax.rsqrt(v + 1e-5) * g + b


def _full(shape):
    n = len(shape)
    return pl.BlockSpec(shape, lambda *_: (0,) * n)


def _att_narrow_body(th, nw, agts, actr, ctx, cctr, wqT, qg, qb, agtwT, dw0T,
                     db0, d1T, d1g, d1b, md, c0g, c0b, mq, mc, mc1, ng, nb2,
                     mlin, lg, lb, out):
    a = agts[...]
    bn = a.shape[0]
    q = jax.nn.relu(_ln(jnp.dot(a, wqT[...]), qg[...], qb[...]))
    qc = jnp.dot(q, mq[...])
    base = jnp.dot(a, agtwT[...])
    ac = actr[...]
    cc = cctr[...]
    dt = dw0T[...]
    b0 = db0[...]
    Ab = ac[:, 0:1] * dt[0:1, :] + ac[:, 1:2] * dt[1:2, :]
    d1Tv, d1gv, d1bv = d1T[...], d1g[...], d1b[...]
    mdv, c0gv, c0bv = md[...], c0g[...], c0b[...]
    mcv = mc[...]

    def body(w, acc):
        crow = cctr[pl.ds(w, 1), :]
        cwrow = crow[:, 0:1] * dt[0:1, :] + crow[:, 1:2] * dt[1:2, :]
        ctxrow = jnp.dot(ctx[pl.ds(w, 1), :], mcv)
        d0 = jax.nn.relu(Ab - cwrow + b0)
        d1 = jax.nn.relu(_ln(jnp.dot(d0, d1Tv), d1gv, d1bv))
        h = jnp.dot(d1, mdv) + qc + ctxrow
        h = jax.nn.relu(_ln(h, c0gv, c0bv))
        dx = ac - crow
        dsq = dx[:, 0:1] * dx[:, 0:1] + dx[:, 1:2] * dx[:, 1:2]
        msk = jnp.sqrt(dsq) <= th
        return acc + jnp.where(msk, h, 0.0)

    acc = lax.fori_loop(0, nw, body, jnp.zeros((bn, D), jnp.float32))
    o = base + jnp.dot(acc, mc1[...])
    o = jax.nn.relu(_ln(o, ng[...], nb2[...]))
    o = _ln(jnp.dot(o, mlin[...]), lg[...], lb[...])
    out[...] = jax.nn.relu(o + a)


def _att_narrow(th, agts, actr, ctx, cctr, w, bn):
    n = agts.shape[0]
    nw = ctx.shape[0]
    grid = n // bn
    wnames = ("wqT", "qg", "qb", "agtwT", "dw0T", "db0", "d1T", "d1g", "d1b",
              "md", "c0g", "c0b", "mq", "mc", "mc1", "ng", "nb2", "mlin",
              "lg", "lb")
    wspecs = [_full(w[k].shape) for k in wnames]
    return pl.pallas_call(
        functools.partial(_att_narrow_body, th, nw),
        grid=(grid,),
        in_specs=[
            pl.BlockSpec((bn, D), lambda i: (i, 0)),
            pl.BlockSpec((bn, 2), lambda i: (i, 0)),
            _full((nw, D)),
            _full((nw, 2)),
        ] + wspecs,
        out_specs=pl.BlockSpec((bn, D), lambda i: (i, 0)),
        out_shape=jax.ShapeDtypeStruct((n, D), jnp.float32),
    )(agts, actr, ctx, cctr, *[w[k] for k in wnames])


def _att_wide_body(th, ncb, agts, actr, ctx, cctr, wqT, qg, qb, agtwT, dw0T,
                   db0, d1T, d1g, d1b, md, c0g, c0b, mq, mc, mc1, ng, nb2,
                   mlin, lg, lb, out, acc):
    k = pl.program_id(0)

    @pl.when(k == 0)
    def _():
        acc[...] = jnp.zeros((NA, D), jnp.float32)

    a = agts[...]
    ac = actr[...]
    cc = cctr[...]
    dt = dw0T[...]
    b0 = db0[...]
    q = jax.nn.relu(_ln(jnp.dot(a, wqT[...]), qg[...], qb[...]))
    qc = jnp.dot(q, mq[...])
    Ab = ac[:, 0:1] * dt[0:1, :] + ac[:, 1:2] * dt[1:2, :]
    Cw = cc[:, 0:1] * dt[0:1, :] + cc[:, 1:2] * dt[1:2, :]
    ctxc = jnp.dot(ctx[...], mc[...])

    P = NA * CB
    d0 = jax.nn.relu(Ab[:, None, :] - Cw[None, :, :] + b0[None, :, :])
    d0 = d0.reshape(P, D)
    d1 = jax.nn.relu(_ln(jnp.dot(d0, d1T[...]), d1g[...], d1b[...]))
    qcp = jnp.broadcast_to(qc[:, None, :], (NA, CB, D)).reshape(P, D)
    ctxp = jnp.broadcast_to(ctxc[None, :, :], (NA, CB, D)).reshape(P, D)
    h = jnp.dot(d1, md[...]) + qcp + ctxp
    h = jax.nn.relu(_ln(h, c0g[...], c0b[...]))
    dxy = (ac[:, None, :] - cc[None, :, :]).reshape(P, 2)
    dsq = dxy[:, 0:1] * dxy[:, 0:1] + dxy[:, 1:2] * dxy[:, 1:2]
    msk = jnp.sqrt(dsq) <= th
    mh = jnp.where(msk, h, 0.0)
    r0 = lax.broadcasted_iota(jnp.int32, (NA, P), 0)
    r1 = lax.broadcasted_iota(jnp.int32, (NA, P), 1)
    R = jnp.where(r1 // CB == r0, 1.0, 0.0)
    acc[...] += jnp.dot(R, mh)

    @pl.when(k == ncb - 1)
    def _():
        base = jnp.dot(a, agtwT[...])
        o = base + jnp.dot(acc[...], mc1[...])
        o = jax.nn.relu(_ln(o, ng[...], nb2[...]))
        o = _ln(jnp.dot(o, mlin[...]), lg[...], lb[...])
        out[...] = jax.nn.relu(o + a)


def _att_wide(th, agts, actr, ctx, cctr, w):
    nc = ctx.shape[0]
    ncb = nc // CB
    wnames = ("wqT", "qg", "qb", "agtwT", "dw0T", "db0", "d1T", "d1g", "d1b",
              "md", "c0g", "c0b", "mq", "mc", "mc1", "ng", "nb2", "mlin",
              "lg", "lb")
    wspecs = [_full(w[k].shape) for k in wnames]
    return pl.pallas_call(
        functools.partial(_att_wide_body, th, ncb),
        grid=(ncb,),
        in_specs=[
            _full((NA, D)),
            _full((NA, 2)),
            pl.BlockSpec((CB, D), lambda i: (i, 0)),
            pl.BlockSpec((CB, 2), lambda i: (i, 0)),
        ] + wspecs,
        out_specs=_full((NA, D)),
        out_shape=jax.ShapeDtypeStruct((NA, D), jnp.float32),
        scratch_shapes=[pltpu.VMEM((NA, D), jnp.float32)],
    )(agts, actr, ctx, cctr, *[w[k] for k in wnames])


def _m2m_pre_body(feat, wT, out):
    out[...] = jnp.dot(feat[...], wT[0])[None]


def _m2m_pre(feat, wT):
    n = feat.shape[0]
    nb = n // BN
    return pl.pallas_call(
        _m2m_pre_body,
        grid=(5, nb),
        in_specs=[
            pl.BlockSpec((BN, D), lambda e, i: (i, 0)),
            pl.BlockSpec((1, D, D), lambda e, i: (e, 0, 0)),
        ],
        out_specs=pl.BlockSpec((1, BN, D), lambda e, i: (e, i, 0)),
        out_shape=jax.ShapeDtypeStruct((5, n, D), jnp.float32),
    )(feat, wT)


def _edge_agg_body(npad, nbatch, rows_per_sub, G, u, v, z, out, idx_u, idx_v,
                   rows, shared):
    cid = lax.axis_index("c")
    sid = lax.axis_index("s")
    wid = cid * 16 + sid
    rbase = pl.multiple_of(sid * rows_per_sub, 8)
    pltpu.sync_copy(z, shared.at[pl.ds(rbase, rows_per_sub)])
    plsc.subcore_barrier()
    ebase = wid * (nbatch * EB)

    def body(b, carry):
        off = pl.multiple_of(ebase + b * EB, EB)
        pltpu.sync_copy(u.at[pl.ds(off, EB)], idx_u)
        pltpu.sync_copy(v.at[pl.ds(off, EB)], idx_v)
        pltpu.sync_copy(G.at[idx_v], rows)
        pltpu.sync_copy(rows, shared.at[idx_u], add=True)
        return carry

    lax.fori_loop(0, nbatch, body, 0)
    plsc.subcore_barrier()
    obase = pl.multiple_of(cid * npad + rbase, 8)
    pltpu.sync_copy(shared.at[pl.ds(rbase, rows_per_sub)],
                    out.at[pl.ds(obase, rows_per_sub)])


def _edge_agg(G, u, v, z, npad, nbatch, rows_per_sub):
    mesh = plsc.VectorSubcoreMesh(core_axis_name="c", subcore_axis_name="s")
    return pl.kernel(
        functools.partial(_edge_agg_body, npad, nbatch, rows_per_sub),
        out_type=jax.ShapeDtypeStruct((2 * npad, D), jnp.float32),
        mesh=mesh,
        scratch_types=[
            pltpu.VMEM((EB,), jnp.int32),
            pltpu.VMEM((EB,), jnp.int32),
            pltpu.VMEM((EB, D), jnp.float32),
            pltpu.VMEM_SHARED((npad, D), jnp.float32),
        ],
    )(G, u, v, z)


def _m2m_post_body(base, agg, feat, ng, nb2, c2T, c2g, c2b, out):
    temp = base[0] + agg[0] + agg[1]
    x = jax.nn.relu(_ln(temp, ng[...], nb2[...]))
    y = _ln(jnp.dot(x, c2T[...]), c2g[...], c2b[...])
    out[...] = jax.nn.relu(y + feat[...])


def _m2m_post(base, agg, feat, ng, nb2, c2T, c2g, c2b):
    n = feat.shape[0]
    nb = n // BN
    return pl.pallas_call(
        _m2m_post_body,
        grid=(nb,),
        in_specs=[
            pl.BlockSpec((1, BN, D), lambda i: (0, i, 0)),
            pl.BlockSpec((2, BN, D), lambda i: (0, i, 0)),
            pl.BlockSpec((BN, D), lambda i: (i, 0)),
            _full((1, D)), _full((1, D)), _full((D, D)), _full((1, D)),
            _full((1, D)),
        ],
        out_specs=pl.BlockSpec((BN, D), lambda i: (i, 0)),
        out_shape=jax.ShapeDtypeStruct((n, D), jnp.float32),
    )(base, agg, feat, ng, nb2, c2T, c2g, c2b)


def _meta_body(x, wT, g, b, out):
    out[...] = jax.nn.relu(_ln(jnp.dot(x[...], wT[...]), g[...], b[...]))


def _meta_lin(x, wT, g, b):
    n = x.shape[0]
    kdim = x.shape[1]
    nb = n // BN
    return pl.pallas_call(
        _meta_body,
        grid=(nb,),
        in_specs=[pl.BlockSpec((BN, kdim), lambda i: (i, 0)),
                  _full((kdim, D)), _full((1, D)), _full((1, D))],
        out_specs=pl.BlockSpec((BN, D), lambda i: (i, 0)),
        out_shape=jax.ShapeDtypeStruct((n, D), jnp.float32),
    )(x, wT, g, b)


def _row(x):
    return x.reshape(1, -1)


def _prep_att(p):
    c0 = p["ctx0"]["w"]
    return {
        "wqT": p["query"]["w"].T, "qg": _row(p["query"]["g"]),
        "qb": _row(p["query"]["b"]), "agtwT": p["agt_w"].T,
        "dw0T": p["dist_w0"].T, "db0": _row(p["dist_b0"]),
        "d1T": p["dist1"]["w"].T, "d1g": _row(p["dist1"]["g"]),
        "d1b": _row(p["dist1"]["b"]),
        "md": c0[:, :D].T, "mq": c0[:, D:2 * D].T, "mc": c0[:, 2 * D:].T,
        "c0g": _row(p["ctx0"]["g"]), "c0b": _row(p["ctx0"]["b"]),
        "mc1": p["ctx1_w"].T,
        "ng": _row(p["norm_g"]), "nb2": _row(p["norm_b"]),
        "mlin": p["lin"]["w"].T, "lg": _row(p["lin"]["g"]),
        "lb": _row(p["lin"]["b"]),
    }


def kernel(actors, nodes, actor_ctrs, graph_ctrs, graph_turn, graph_control,
           graph_intersect, pre_u, pre_v, suc_u, suc_v, left_u, left_v,
           right_u, right_v, params):
    n = nodes.shape[0]
    npad = ((n + BN - 1) // BN) * BN
    padn = npad - n

    nodes_p = jnp.pad(nodes, ((0, padn), (0, 0)))
    turn_p = jnp.pad(graph_turn, ((0, padn), (0, 0)))
    ctrl_p = jnp.pad(graph_control, (0, padn))[:, None]
    inter_p = jnp.pad(graph_intersect, (0, padn))[:, None]
    gctr_p = jnp.pad(graph_ctrs, ((0, padn), (0, 0)), constant_values=1e9)

    xcat = jnp.concatenate([nodes_p, turn_p, ctrl_p, inter_p], axis=1)
    kdim = 256
    xcat = jnp.pad(xcat, ((0, 0), (0, kdim - xcat.shape[1])))
    mw = params["a2m_meta"]
    metaT = jnp.pad(mw["w"].T, ((0, kdim - mw["w"].shape[1]), (0, 0)))
    feat = _meta_lin(xcat, metaT, _row(mw["g"]), _row(mw["b"]))
    for i in range(2):
        feat = _att_narrow(7.0, feat, gctr_p, actors, actor_ctrs,
                           _prep_att(params["a2m_att"][i]), BN)

    e_real = 2 * pre_u.shape[0] + 2 * left_u.shape[0]
    per_w = ((e_real + NWORK * EB - 1) // (NWORK * EB)) * EB
    e_pad = NWORK * per_w
    nbatch = per_w // EB
    u_all = jnp.concatenate([pre_u, suc_u, left_u, right_u])
    v_off = jnp.concatenate([pre_v + npad, suc_v + 2 * npad,
                             left_v + 3 * npad, right_v + 4 * npad])
    u_all = jnp.pad(u_all, (0, e_pad - e_real), constant_values=npad - 1)
    v_off = jnp.pad(v_off, (0, e_pad - e_real))
    rows_per_sub = npad // 16
    zrows = jnp.zeros((rows_per_sub, D), jnp.float32)

    m = params["m2m"]
    res = feat
    for i in range(4):
        wT = jnp.stack([m["ctr"][i].T, m["pre"][i].T, m["suc"][i].T,
                        m["left"][i].T, m["right"][i].T])
        G = _m2m_pre(feat, wT)
        agg = _edge_agg(G.reshape(5 * npad, D), u_all, v_off, zrows,
                        npad, nbatch, rows_per_sub)
        feat = _m2m_post(G[0:1], agg.reshape(2, npad, D), res,
                         _row(m["norm_g"][i]), _row(m["norm_b"][i]),
                         m["ctr2_w"][i].T, _row(m["ctr2_g"][i]),
                         _row(m["ctr2_b"][i]))
        res = feat

    acts = actors
    for i in range(2):
        acts = _att_wide(6.0, acts, actor_ctrs, feat, gctr_p,
                         _prep_att(params["m2a_att"][i]))

    for i in range(2):
        acts = _att_narrow(100.0, acts, actor_ctrs, acts, actor_ctrs,
                           _prep_att(params["a2a_att"][i]), NA)

    return acts


# speedup vs baseline: 7.4316x; 1.5339x over previous
import functools

import jax
import jax.numpy as jnp
from jax import lax
from jax.experimental import pallas as pl
from jax.experimental.pallas import tpu as pltpu
from jax.experimental.pallas import tpu_sc as plsc

D = 128
NA = 64
BN = 256
BA = 128
CB = 128
EB = 128
NWORK = 32


def _ln(x, g, b):
    m = jnp.mean(x, axis=-1, keepdims=True)
    d = x - m
    v = jnp.mean(d * d, axis=-1, keepdims=True)
    return d * lax.rsqrt(v + 1e-5) * g + b


def _full(shape):
    n = len(shape)
    return pl.BlockSpec(shape, lambda *_: (0,) * n)


def _pair_sum(th, nw, bn, Ab, qc, ac, Cw, ctxc, cc, b0, d1T, d1g, d1b, md,
              c0g, c0b):
    P = nw * bn
    d0 = jax.nn.relu((Ab[None, :, :] - Cw[:, None, :]
                      + b0[None, :, :]).reshape(P, D))
    d1 = jax.nn.relu(_ln(jnp.dot(d0, d1T), d1g, d1b))
    qcp = jnp.broadcast_to(qc[None, :, :], (nw, bn, D)).reshape(P, D)
    cxp = jnp.broadcast_to(ctxc[:, None, :], (nw, bn, D)).reshape(P, D)
    h = jax.nn.relu(_ln(jnp.dot(d1, md) + qcp + cxp, c0g, c0b))
    dxy = (ac[None, :, :] - cc[:, None, :]).reshape(P, 2)
    dsq = dxy[:, 0:1] * dxy[:, 0:1] + dxy[:, 1:2] * dxy[:, 1:2]
    mh = jnp.where(jnp.sqrt(dsq) <= th, h, 0.0)
    m = nw
    while m > 1:
        m //= 2
        mh = mh[: m * bn] + mh[m * bn:]
    return mh


def _proj2(xy, dt):
    return xy[:, 0:1] * dt[0:1, :] + xy[:, 1:2] * dt[1:2, :]


def _att_narrow_body(th, nw, agts, actr, ctx, cctr, wqT, qg, qb, agtwT, dw0T,
                     db0, d1T, d1g, d1b, md, c0g, c0b, mq, mc, mc1, ng, nb2,
                     mlin, lg, lb, out):
    a = agts[...]
    bn = a.shape[0]
    q = jax.nn.relu(_ln(jnp.dot(a, wqT[...]), qg[...], qb[...]))
    qc = jnp.dot(q, mq[...])
    base = jnp.dot(a, agtwT[...])
    ac = actr[...]
    cc = cctr[...]
    dt = dw0T[...]
    Ab = _proj2(ac, dt)
    Cw = _proj2(cc, dt)
    ctxc = jnp.dot(ctx[...], mc[...])
    acc = _pair_sum(th, nw, bn, Ab, qc, ac, Cw, ctxc, cc, db0[...],
                    d1T[...], d1g[...], d1b[...], md[...], c0g[...],
                    c0b[...])
    o = base + jnp.dot(acc, mc1[...])
    o = jax.nn.relu(_ln(o, ng[...], nb2[...]))
    o = _ln(jnp.dot(o, mlin[...]), lg[...], lb[...])
    out[...] = jax.nn.relu(o + a)


def _att_narrow(th, agts, actr, ctx, cctr, w, bn):
    n = agts.shape[0]
    nw = ctx.shape[0]
    grid = n // bn
    wnames = ("wqT", "qg", "qb", "agtwT", "dw0T", "db0", "d1T", "d1g", "d1b",
              "md", "c0g", "c0b", "mq", "mc", "mc1", "ng", "nb2", "mlin",
              "lg", "lb")
    wspecs = [_full(w[k].shape) for k in wnames]
    return pl.pallas_call(
        functools.partial(_att_narrow_body, th, nw),
        grid=(grid,),
        in_specs=[
            pl.BlockSpec((bn, D), lambda i: (i, 0)),
            pl.BlockSpec((bn, 2), lambda i: (i, 0)),
            _full((nw, D)),
            _full((nw, 2)),
        ] + wspecs,
        out_specs=pl.BlockSpec((bn, D), lambda i: (i, 0)),
        out_shape=jax.ShapeDtypeStruct((n, D), jnp.float32),
    )(agts, actr, ctx, cctr, *[w[k] for k in wnames])


def _att_wide_body(th, ncb, agts, actr, ctx, cctr, wqT, qg, qb, agtwT, dw0T,
                   db0, d1T, d1g, d1b, md, c0g, c0b, mq, mc, mc1, ng, nb2,
                   mlin, lg, lb, out, acc):
    k = pl.program_id(0)

    @pl.when(k == 0)
    def _():
        acc[...] = jnp.zeros((NA, D), jnp.float32)

    a = agts[...]
    ac = actr[...]
    cc = cctr[...]
    dt = dw0T[...]
    q = jax.nn.relu(_ln(jnp.dot(a, wqT[...]), qg[...], qb[...]))
    qc = jnp.dot(q, mq[...])
    Ab = _proj2(ac, dt)
    Cw = _proj2(cc, dt)
    ctxc = jnp.dot(ctx[...], mc[...])
    acc[...] += _pair_sum(th, CB, NA, Ab, qc, ac, Cw, ctxc, cc, db0[...],
                          d1T[...], d1g[...], d1b[...], md[...], c0g[...],
                          c0b[...])

    @pl.when(k == ncb - 1)
    def _():
        base = jnp.dot(a, agtwT[...])
        o = base + jnp.dot(acc[...], mc1[...])
        o = jax.nn.relu(_ln(o, ng[...], nb2[...]))
        o = _ln(jnp.dot(o, mlin[...]), lg[...], lb[...])
        out[...] = jax.nn.relu(o + a)


def _att_wide(th, agts, actr, ctx, cctr, w):
    nc = ctx.shape[0]
    ncb = nc // CB
    wnames = ("wqT", "qg", "qb", "agtwT", "dw0T", "db0", "d1T", "d1g", "d1b",
              "md", "c0g", "c0b", "mq", "mc", "mc1", "ng", "nb2", "mlin",
              "lg", "lb")
    wspecs = [_full(w[k].shape) for k in wnames]
    return pl.pallas_call(
        functools.partial(_att_wide_body, th, ncb),
        grid=(ncb,),
        in_specs=[
            _full((NA, D)),
            _full((NA, 2)),
            pl.BlockSpec((CB, D), lambda i: (i, 0)),
            pl.BlockSpec((CB, 2), lambda i: (i, 0)),
        ] + wspecs,
        out_specs=_full((NA, D)),
        out_shape=jax.ShapeDtypeStruct((NA, D), jnp.float32),
        scratch_shapes=[pltpu.VMEM((NA, D), jnp.float32)],
    )(agts, actr, ctx, cctr, *[w[k] for k in wnames])


def _m2m_pre_body(feat, wT, out):
    out[...] = jnp.dot(feat[...], wT[0])[None]


def _m2m_pre(feat, wT):
    n = feat.shape[0]
    nb = n // BN
    return pl.pallas_call(
        _m2m_pre_body,
        grid=(5, nb),
        in_specs=[
            pl.BlockSpec((BN, D), lambda e, i: (i, 0)),
            pl.BlockSpec((1, D, D), lambda e, i: (e, 0, 0)),
        ],
        out_specs=pl.BlockSpec((1, BN, D), lambda e, i: (e, i, 0)),
        out_shape=jax.ShapeDtypeStruct((5, n, D), jnp.float32),
    )(feat, wT)


def _edge_agg_body(npad, nbatch, rows_per_sub, G, u, v, z, out, idx_u, idx_v,
                   rows, shared):
    cid = lax.axis_index("c")
    sid = lax.axis_index("s")
    wid = cid * 16 + sid
    rbase = pl.multiple_of(sid * rows_per_sub, 8)
    pltpu.sync_copy(z, shared.at[pl.ds(rbase, rows_per_sub)])
    plsc.subcore_barrier()
    ebase = wid * (nbatch * EB)

    def body(b, carry):
        off = pl.multiple_of(ebase + b * EB, EB)
        pltpu.sync_copy(u.at[pl.ds(off, EB)], idx_u)
        pltpu.sync_copy(v.at[pl.ds(off, EB)], idx_v)
        pltpu.sync_copy(G.at[idx_v], rows)
        pltpu.sync_copy(rows, shared.at[idx_u], add=True)
        return carry

    lax.fori_loop(0, nbatch, body, 0)
    plsc.subcore_barrier()
    obase = pl.multiple_of(cid * npad + rbase, 8)
    pltpu.sync_copy(shared.at[pl.ds(rbase, rows_per_sub)],
                    out.at[pl.ds(obase, rows_per_sub)])


def _edge_agg(G, u, v, z, npad, nbatch, rows_per_sub):
    mesh = plsc.VectorSubcoreMesh(core_axis_name="c", subcore_axis_name="s")
    return pl.kernel(
        functools.partial(_edge_agg_body, npad, nbatch, rows_per_sub),
        out_type=jax.ShapeDtypeStruct((2 * npad, D), jnp.float32),
        mesh=mesh,
        scratch_types=[
            pltpu.VMEM((EB,), jnp.int32),
            pltpu.VMEM((EB,), jnp.int32),
            pltpu.VMEM((EB, D), jnp.float32),
            pltpu.VMEM_SHARED((npad, D), jnp.float32),
        ],
    )(G, u, v, z)


def _m2m_post_body(base, agg, feat, ng, nb2, c2T, c2g, c2b, out):
    temp = base[0] + agg[0] + agg[1]
    x = jax.nn.relu(_ln(temp, ng[...], nb2[...]))
    y = _ln(jnp.dot(x, c2T[...]), c2g[...], c2b[...])
    out[...] = jax.nn.relu(y + feat[...])


def _m2m_post(base, agg, feat, ng, nb2, c2T, c2g, c2b):
    n = feat.shape[0]
    nb = n // BN
    return pl.pallas_call(
        _m2m_post_body,
        grid=(nb,),
        in_specs=[
            pl.BlockSpec((1, BN, D), lambda i: (0, i, 0)),
            pl.BlockSpec((2, BN, D), lambda i: (0, i, 0)),
            pl.BlockSpec((BN, D), lambda i: (i, 0)),
            _full((1, D)), _full((1, D)), _full((D, D)), _full((1, D)),
            _full((1, D)),
        ],
        out_specs=pl.BlockSpec((BN, D), lambda i: (i, 0)),
        out_shape=jax.ShapeDtypeStruct((n, D), jnp.float32),
    )(base, agg, feat, ng, nb2, c2T, c2g, c2b)


def _meta_body(x, wT, g, b, out):
    out[...] = jax.nn.relu(_ln(jnp.dot(x[...], wT[...]), g[...], b[...]))


def _meta_lin(x, wT, g, b):
    n = x.shape[0]
    kdim = x.shape[1]
    nb = n // BN
    return pl.pallas_call(
        _meta_body,
        grid=(nb,),
        in_specs=[pl.BlockSpec((BN, kdim), lambda i: (i, 0)),
                  _full((kdim, D)), _full((1, D)), _full((1, D))],
        out_specs=pl.BlockSpec((BN, D), lambda i: (i, 0)),
        out_shape=jax.ShapeDtypeStruct((n, D), jnp.float32),
    )(x, wT, g, b)


def _row(x):
    return x.reshape(1, -1)


def _prep_att(p):
    c0 = p["ctx0"]["w"]
    return {
        "wqT": p["query"]["w"].T, "qg": _row(p["query"]["g"]),
        "qb": _row(p["query"]["b"]), "agtwT": p["agt_w"].T,
        "dw0T": p["dist_w0"].T, "db0": _row(p["dist_b0"]),
        "d1T": p["dist1"]["w"].T, "d1g": _row(p["dist1"]["g"]),
        "d1b": _row(p["dist1"]["b"]),
        "md": c0[:, :D].T, "mq": c0[:, D:2 * D].T, "mc": c0[:, 2 * D:].T,
        "c0g": _row(p["ctx0"]["g"]), "c0b": _row(p["ctx0"]["b"]),
        "mc1": p["ctx1_w"].T,
        "ng": _row(p["norm_g"]), "nb2": _row(p["norm_b"]),
        "mlin": p["lin"]["w"].T, "lg": _row(p["lin"]["g"]),
        "lb": _row(p["lin"]["b"]),
    }


def kernel(actors, nodes, actor_ctrs, graph_ctrs, graph_turn, graph_control,
           graph_intersect, pre_u, pre_v, suc_u, suc_v, left_u, left_v,
           right_u, right_v, params):
    n = nodes.shape[0]
    npad = ((n + BN - 1) // BN) * BN
    padn = npad - n

    nodes_p = jnp.pad(nodes, ((0, padn), (0, 0)))
    turn_p = jnp.pad(graph_turn, ((0, padn), (0, 0)))
    ctrl_p = jnp.pad(graph_control, (0, padn))[:, None]
    inter_p = jnp.pad(graph_intersect, (0, padn))[:, None]
    gctr_p = jnp.pad(graph_ctrs, ((0, padn), (0, 0)), constant_values=1e9)

    xcat = jnp.concatenate([nodes_p, turn_p, ctrl_p, inter_p], axis=1)
    kdim = 256
    xcat = jnp.pad(xcat, ((0, 0), (0, kdim - xcat.shape[1])))
    mw = params["a2m_meta"]
    metaT = jnp.pad(mw["w"].T, ((0, kdim - mw["w"].shape[1]), (0, 0)))
    feat = _meta_lin(xcat, metaT, _row(mw["g"]), _row(mw["b"]))
    for i in range(2):
        feat = _att_narrow(7.0, feat, gctr_p, actors, actor_ctrs,
                           _prep_att(params["a2m_att"][i]), BA)

    e_real = 2 * pre_u.shape[0] + 2 * left_u.shape[0]
    per_w = ((e_real + NWORK * EB - 1) // (NWORK * EB)) * EB
    e_pad = NWORK * per_w
    nbatch = per_w // EB
    u_all = jnp.concatenate([pre_u, suc_u, left_u, right_u])
    v_off = jnp.concatenate([pre_v + npad, suc_v + 2 * npad,
                             left_v + 3 * npad, right_v + 4 * npad])
    u_all = jnp.pad(u_all, (0, e_pad - e_real), constant_values=npad - 1)
    v_off = jnp.pad(v_off, (0, e_pad - e_real))
    rows_per_sub = npad // 16
    zrows = jnp.zeros((rows_per_sub, D), jnp.float32)

    m = params["m2m"]
    res = feat
    for i in range(4):
        wT = jnp.stack([m["ctr"][i].T, m["pre"][i].T, m["suc"][i].T,
                        m["left"][i].T, m["right"][i].T])
        G = _m2m_pre(feat, wT)
        agg = _edge_agg(G.reshape(5 * npad, D), u_all, v_off, zrows,
                        npad, nbatch, rows_per_sub)
        feat = _m2m_post(G[0:1], agg.reshape(2, npad, D), res,
                         _row(m["norm_g"][i]), _row(m["norm_b"][i]),
                         m["ctr2_w"][i].T, _row(m["ctr2_g"][i]),
                         _row(m["ctr2_b"][i]))
        res = feat

    acts = actors
    for i in range(2):
        acts = _att_wide(6.0, acts, actor_ctrs, feat, gctr_p,
                         _prep_att(params["m2a_att"][i]))

    for i in range(2):
        acts = _att_narrow(100.0, acts, actor_ctrs, acts, actor_ctrs,
                           _prep_att(params["a2a_att"][i]), NA)

    return acts


# speedup vs baseline: 7.7332x; 1.0406x over previous
import functools

import jax
import jax.numpy as jnp
from jax import lax
from jax.experimental import pallas as pl
from jax.experimental.pallas import tpu as pltpu
from jax.experimental.pallas import tpu_sc as plsc

D = 128
NA = 64
BN = 256
BA = 128
CB = 128
EB = 128
NWORK = 32


def _ln(x, g, b):
    m = jnp.mean(x, axis=-1, keepdims=True)
    d = x - m
    v = jnp.mean(d * d, axis=-1, keepdims=True)
    return d * lax.rsqrt(v + 1e-5) * g + b


def _full(shape):
    n = len(shape)
    return pl.BlockSpec(shape, lambda *_: (0,) * n)


def _pair_sum(th, nw, bn, Ab, qc, ac, Cw, ctxc, cc, b0, d1T, d1g, d1b, md,
              c0g, c0b):
    P = nw * bn
    d0 = jax.nn.relu((Ab[None, :, :] - Cw[:, None, :]
                      + b0[None, :, :]).reshape(P, D))
    d1 = jax.nn.relu(_ln(jnp.dot(d0, d1T), d1g, d1b))
    qcp = jnp.broadcast_to(qc[None, :, :], (nw, bn, D)).reshape(P, D)
    cxp = jnp.broadcast_to(ctxc[:, None, :], (nw, bn, D)).reshape(P, D)
    h = jax.nn.relu(_ln(jnp.dot(d1, md) + qcp + cxp, c0g, c0b))
    dxy = (ac[None, :, :] - cc[:, None, :]).reshape(P, 2)
    dsq = dxy[:, 0:1] * dxy[:, 0:1] + dxy[:, 1:2] * dxy[:, 1:2]
    mh = jnp.where(jnp.sqrt(dsq) <= th, h, 0.0)
    m = nw
    while m > 1:
        m //= 2
        mh = mh[: m * bn] + mh[m * bn:]
    return mh


def _proj2(xy, dt):
    return xy[:, 0:1] * dt[0:1, :] + xy[:, 1:2] * dt[1:2, :]


def _att_narrow_body(th, nw, agts, actr, ctx, cctr, wqT, qg, qb, agtwT, dw0T,
                     db0, d1T, d1g, d1b, md, c0g, c0b, mq, mc, mc1, ng, nb2,
                     mlin, lg, lb, out):
    a = agts[...]
    bn = a.shape[0]
    q = jax.nn.relu(_ln(jnp.dot(a, wqT[...]), qg[...], qb[...]))
    qc = jnp.dot(q, mq[...])
    base = jnp.dot(a, agtwT[...])
    ac = actr[...]
    cc = cctr[...]
    dt = dw0T[...]
    Ab = _proj2(ac, dt)
    Cw = _proj2(cc, dt)
    ctxc = jnp.dot(ctx[...], mc[...])
    acc = _pair_sum(th, nw, bn, Ab, qc, ac, Cw, ctxc, cc, db0[...],
                    d1T[...], d1g[...], d1b[...], md[...], c0g[...],
                    c0b[...])
    o = base + jnp.dot(acc, mc1[...])
    o = jax.nn.relu(_ln(o, ng[...], nb2[...]))
    o = _ln(jnp.dot(o, mlin[...]), lg[...], lb[...])
    out[...] = jax.nn.relu(o + a)


def _att_narrow(th, agts, actr, ctx, cctr, w, bn):
    n = agts.shape[0]
    nw = ctx.shape[0]
    grid = n // bn
    wnames = ("wqT", "qg", "qb", "agtwT", "dw0T", "db0", "d1T", "d1g", "d1b",
              "md", "c0g", "c0b", "mq", "mc", "mc1", "ng", "nb2", "mlin",
              "lg", "lb")
    wspecs = [_full(w[k].shape) for k in wnames]
    return pl.pallas_call(
        functools.partial(_att_narrow_body, th, nw),
        grid=(grid,),
        in_specs=[
            pl.BlockSpec((bn, D), lambda i: (i, 0)),
            pl.BlockSpec((bn, 2), lambda i: (i, 0)),
            _full((nw, D)),
            _full((nw, 2)),
        ] + wspecs,
        out_specs=pl.BlockSpec((bn, D), lambda i: (i, 0)),
        out_shape=jax.ShapeDtypeStruct((n, D), jnp.float32),
    )(agts, actr, ctx, cctr, *[w[k] for k in wnames])


def _att_wide_body(th, ncb, agts, actr, ctx, cctr, wqT, qg, qb, agtwT, dw0T,
                   db0, d1T, d1g, d1b, md, c0g, c0b, mq, mc, mc1, ng, nb2,
                   mlin, lg, lb, out, acc):
    k = pl.program_id(0)

    @pl.when(k == 0)
    def _():
        acc[...] = jnp.zeros((NA, D), jnp.float32)

    a = agts[...]
    ac = actr[...]
    cc = cctr[...]
    dt = dw0T[...]
    q = jax.nn.relu(_ln(jnp.dot(a, wqT[...]), qg[...], qb[...]))
    qc = jnp.dot(q, mq[...])
    Ab = _proj2(ac, dt)
    Cw = _proj2(cc, dt)
    ctxc = jnp.dot(ctx[...], mc[...])
    acc[...] += _pair_sum(th, CB, NA, Ab, qc, ac, Cw, ctxc, cc, db0[...],
                          d1T[...], d1g[...], d1b[...], md[...], c0g[...],
                          c0b[...])

    @pl.when(k == ncb - 1)
    def _():
        base = jnp.dot(a, agtwT[...])
        o = base + jnp.dot(acc[...], mc1[...])
        o = jax.nn.relu(_ln(o, ng[...], nb2[...]))
        o = _ln(jnp.dot(o, mlin[...]), lg[...], lb[...])
        out[...] = jax.nn.relu(o + a)


def _att_wide(th, agts, actr, ctx, cctr, w):
    nc = ctx.shape[0]
    ncb = nc // CB
    wnames = ("wqT", "qg", "qb", "agtwT", "dw0T", "db0", "d1T", "d1g", "d1b",
              "md", "c0g", "c0b", "mq", "mc", "mc1", "ng", "nb2", "mlin",
              "lg", "lb")
    wspecs = [_full(w[k].shape) for k in wnames]
    return pl.pallas_call(
        functools.partial(_att_wide_body, th, ncb),
        grid=(ncb,),
        in_specs=[
            _full((NA, D)),
            _full((NA, 2)),
            pl.BlockSpec((CB, D), lambda i: (i, 0)),
            pl.BlockSpec((CB, 2), lambda i: (i, 0)),
        ] + wspecs,
        out_specs=_full((NA, D)),
        out_shape=jax.ShapeDtypeStruct((NA, D), jnp.float32),
        scratch_shapes=[pltpu.VMEM((NA, D), jnp.float32)],
    )(agts, actr, ctx, cctr, *[w[k] for k in wnames])


def _m2m_pre_body(feat, wT, out):
    out[...] = jnp.dot(feat[...], wT[0])[None]


def _m2m_pre(feat, wT):
    n = feat.shape[0]
    nb = n // BN
    return pl.pallas_call(
        _m2m_pre_body,
        grid=(5, nb),
        in_specs=[
            pl.BlockSpec((BN, D), lambda e, i: (i, 0)),
            pl.BlockSpec((1, D, D), lambda e, i: (e, 0, 0)),
        ],
        out_specs=pl.BlockSpec((1, BN, D), lambda e, i: (e, i, 0)),
        out_shape=jax.ShapeDtypeStruct((5, n, D), jnp.float32),
    )(feat, wT)


def _edge_agg_body(npad, nbatch, rows_per_sub, G, u, v, z, out, idx_u, idx_v,
                   rows, shared, gsem, ssem0, ssem1):
    cid = lax.axis_index("c")
    sid = lax.axis_index("s")
    wid = cid * 16 + sid
    rbase = pl.multiple_of(sid * rows_per_sub, 8)
    pltpu.sync_copy(z, shared.at[pl.ds(rbase, rows_per_sub)])
    plsc.subcore_barrier()
    ebase = wid * (nbatch * EB)
    ssems = (ssem0, ssem1)

    def body2(t, carry):
        for s in (0, 1):
            b = 2 * t + s
            off = pl.multiple_of(ebase + b * EB, EB)

            @pl.when(t > 0)
            def _():
                pltpu.make_async_copy(rows.at[s], shared.at[idx_u.at[s]],
                                      ssems[s]).wait()

            pltpu.sync_copy(u.at[pl.ds(off, EB)], idx_u.at[s])
            pltpu.sync_copy(v.at[pl.ds(off, EB)], idx_v.at[s])
            pltpu.async_copy(G.at[idx_v.at[s]], rows.at[s], gsem).wait()
            pltpu.async_copy(rows.at[s], shared.at[idx_u.at[s]], ssems[s],
                             add=True)
        return carry

    lax.fori_loop(0, nbatch // 2, body2, 0)
    for s in (0, 1):
        pltpu.make_async_copy(rows.at[s], shared.at[idx_u.at[s]],
                              ssems[s]).wait()
    plsc.subcore_barrier()
    obase = pl.multiple_of(cid * npad + rbase, 8)
    pltpu.sync_copy(shared.at[pl.ds(rbase, rows_per_sub)],
                    out.at[pl.ds(obase, rows_per_sub)])


def _edge_agg(G, u, v, z, npad, nbatch, rows_per_sub):
    mesh = plsc.VectorSubcoreMesh(core_axis_name="c", subcore_axis_name="s")
    return pl.kernel(
        functools.partial(_edge_agg_body, npad, nbatch, rows_per_sub),
        out_type=jax.ShapeDtypeStruct((2 * npad, D), jnp.float32),
        mesh=mesh,
        scratch_types=[
            pltpu.VMEM((2, EB), jnp.int32),
            pltpu.VMEM((2, EB), jnp.int32),
            pltpu.VMEM((2, EB, D), jnp.float32),
            pltpu.VMEM_SHARED((npad, D), jnp.float32),
            pltpu.SemaphoreType.DMA,
            pltpu.SemaphoreType.DMA,
            pltpu.SemaphoreType.DMA,
        ],
    )(G, u, v, z)


def _m2m_post_body(base, agg, feat, ng, nb2, c2T, c2g, c2b, out):
    temp = base[0] + agg[0] + agg[1]
    x = jax.nn.relu(_ln(temp, ng[...], nb2[...]))
    y = _ln(jnp.dot(x, c2T[...]), c2g[...], c2b[...])
    out[...] = jax.nn.relu(y + feat[...])


def _m2m_post(base, agg, feat, ng, nb2, c2T, c2g, c2b):
    n = feat.shape[0]
    nb = n // BN
    return pl.pallas_call(
        _m2m_post_body,
        grid=(nb,),
        in_specs=[
            pl.BlockSpec((1, BN, D), lambda i: (0, i, 0)),
            pl.BlockSpec((2, BN, D), lambda i: (0, i, 0)),
            pl.BlockSpec((BN, D), lambda i: (i, 0)),
            _full((1, D)), _full((1, D)), _full((D, D)), _full((1, D)),
            _full((1, D)),
        ],
        out_specs=pl.BlockSpec((BN, D), lambda i: (i, 0)),
        out_shape=jax.ShapeDtypeStruct((n, D), jnp.float32),
    )(base, agg, feat, ng, nb2, c2T, c2g, c2b)


def _meta_body(x, wT, g, b, out):
    out[...] = jax.nn.relu(_ln(jnp.dot(x[...], wT[...]), g[...], b[...]))


def _meta_lin(x, wT, g, b):
    n = x.shape[0]
    kdim = x.shape[1]
    nb = n // BN
    return pl.pallas_call(
        _meta_body,
        grid=(nb,),
        in_specs=[pl.BlockSpec((BN, kdim), lambda i: (i, 0)),
                  _full((kdim, D)), _full((1, D)), _full((1, D))],
        out_specs=pl.BlockSpec((BN, D), lambda i: (i, 0)),
        out_shape=jax.ShapeDtypeStruct((n, D), jnp.float32),
    )(x, wT, g, b)


def _row(x):
    return x.reshape(1, -1)


def _prep_att(p):
    c0 = p["ctx0"]["w"]
    return {
        "wqT": p["query"]["w"].T, "qg": _row(p["query"]["g"]),
        "qb": _row(p["query"]["b"]), "agtwT": p["agt_w"].T,
        "dw0T": p["dist_w0"].T, "db0": _row(p["dist_b0"]),
        "d1T": p["dist1"]["w"].T, "d1g": _row(p["dist1"]["g"]),
        "d1b": _row(p["dist1"]["b"]),
        "md": c0[:, :D].T, "mq": c0[:, D:2 * D].T, "mc": c0[:, 2 * D:].T,
        "c0g": _row(p["ctx0"]["g"]), "c0b": _row(p["ctx0"]["b"]),
        "mc1": p["ctx1_w"].T,
        "ng": _row(p["norm_g"]), "nb2": _row(p["norm_b"]),
        "mlin": p["lin"]["w"].T, "lg": _row(p["lin"]["g"]),
        "lb": _row(p["lin"]["b"]),
    }


def kernel(actors, nodes, actor_ctrs, graph_ctrs, graph_turn, graph_control,
           graph_intersect, pre_u, pre_v, suc_u, suc_v, left_u, left_v,
           right_u, right_v, params):
    n = nodes.shape[0]
    npad = ((n + BN - 1) // BN) * BN
    padn = npad - n

    nodes_p = jnp.pad(nodes, ((0, padn), (0, 0)))
    turn_p = jnp.pad(graph_turn, ((0, padn), (0, 0)))
    ctrl_p = jnp.pad(graph_control, (0, padn))[:, None]
    inter_p = jnp.pad(graph_intersect, (0, padn))[:, None]
    gctr_p = jnp.pad(graph_ctrs, ((0, padn), (0, 0)), constant_values=1e9)

    xcat = jnp.concatenate([nodes_p, turn_p, ctrl_p, inter_p], axis=1)
    kdim = 256
    xcat = jnp.pad(xcat, ((0, 0), (0, kdim - xcat.shape[1])))
    mw = params["a2m_meta"]
    metaT = jnp.pad(mw["w"].T, ((0, kdim - mw["w"].shape[1]), (0, 0)))
    feat = _meta_lin(xcat, metaT, _row(mw["g"]), _row(mw["b"]))
    for i in range(2):
        feat = _att_narrow(7.0, feat, gctr_p, actors, actor_ctrs,
                           _prep_att(params["a2m_att"][i]), BA)

    e_real = 2 * pre_u.shape[0] + 2 * left_u.shape[0]
    gran = 2 * EB
    per_w = ((e_real + NWORK * gran - 1) // (NWORK * gran)) * gran
    e_pad = NWORK * per_w
    nbatch = per_w // EB
    u_all = jnp.concatenate([pre_u, suc_u, left_u, right_u])
    v_off = jnp.concatenate([pre_v + npad, suc_v + 2 * npad,
                             left_v + 3 * npad, right_v + 4 * npad])
    u_all = jnp.pad(u_all, (0, e_pad - e_real), constant_values=npad - 1)
    v_off = jnp.pad(v_off, (0, e_pad - e_real))
    rows_per_sub = npad // 16
    zrows = jnp.zeros((rows_per_sub, D), jnp.float32)

    m = params["m2m"]
    res = feat
    for i in range(4):
        wT = jnp.stack([m["ctr"][i].T, m["pre"][i].T, m["suc"][i].T,
                        m["left"][i].T, m["right"][i].T])
        G = _m2m_pre(feat, wT)
        agg = _edge_agg(G.reshape(5 * npad, D), u_all, v_off, zrows,
                        npad, nbatch, rows_per_sub)
        feat = _m2m_post(G[0:1], agg.reshape(2, npad, D), res,
                         _row(m["norm_g"][i]), _row(m["norm_b"][i]),
                         m["ctr2_w"][i].T, _row(m["ctr2_g"][i]),
                         _row(m["ctr2_b"][i]))
        res = feat

    acts = actors
    for i in range(2):
        acts = _att_wide(6.0, acts, actor_ctrs, feat, gctr_p,
                         _prep_att(params["m2a_att"][i]))

    for i in range(2):
        acts = _att_narrow(100.0, acts, actor_ctrs, acts, actor_ctrs,
                           _prep_att(params["a2a_att"][i]), NA)

    return acts


# speedup vs baseline: 8.0418x; 1.0399x over previous
import functools

import jax
import jax.numpy as jnp
from jax import lax
from jax.experimental import pallas as pl
from jax.experimental.pallas import tpu as pltpu
from jax.experimental.pallas import tpu_sc as plsc

D = 128
NA = 64
BN = 256
BA = 128
CB = 128
EB = 128
NWORK = 32


def _ln(x, g, b):
    m = jnp.mean(x, axis=-1, keepdims=True)
    d = x - m
    v = jnp.mean(d * d, axis=-1, keepdims=True)
    return d * lax.rsqrt(v + 1e-5) * g + b


def _full(shape):
    n = len(shape)
    return pl.BlockSpec(shape, lambda *_: (0,) * n)


def _bdot(x, wT):
    return jnp.dot(x.astype(jnp.bfloat16), wT.astype(jnp.bfloat16),
                   preferred_element_type=jnp.float32)


def _pair_sum(th, nw, bn, Ab, qc, ac, Cw, ctxc, cc, b0, d1T, d1g, d1b, md,
              c0g, c0b):
    P = nw * bn
    d0 = jax.nn.relu((Ab[None, :, :] - Cw[:, None, :]
                      + b0[None, :, :]).reshape(P, D))
    d1 = jax.nn.relu(_ln(_bdot(d0, d1T), d1g, d1b))
    qcp = jnp.broadcast_to(qc[None, :, :], (nw, bn, D)).reshape(P, D)
    cxp = jnp.broadcast_to(ctxc[:, None, :], (nw, bn, D)).reshape(P, D)
    h = jax.nn.relu(_ln(_bdot(d1, md) + qcp + cxp, c0g, c0b))
    dxy = (ac[None, :, :] - cc[:, None, :]).reshape(P, 2)
    dsq = dxy[:, 0:1] * dxy[:, 0:1] + dxy[:, 1:2] * dxy[:, 1:2]
    mh = jnp.where(jnp.sqrt(dsq) <= th, h, 0.0)
    m = nw
    while m > 1:
        m //= 2
        mh = mh[: m * bn] + mh[m * bn:]
    return mh


def _proj2(xy, dt):
    return xy[:, 0:1] * dt[0:1, :] + xy[:, 1:2] * dt[1:2, :]


def _att_narrow_body(th, nw, agts, actr, ctx, cctr, wqT, qg, qb, agtwT, dw0T,
                     db0, d1T, d1g, d1b, md, c0g, c0b, mq, mc, mc1, ng, nb2,
                     mlin, lg, lb, out):
    a = agts[...]
    bn = a.shape[0]
    q = jax.nn.relu(_ln(jnp.dot(a, wqT[...]), qg[...], qb[...]))
    qc = jnp.dot(q, mq[...])
    base = jnp.dot(a, agtwT[...])
    ac = actr[...]
    cc = cctr[...]
    dt = dw0T[...]
    Ab = _proj2(ac, dt)
    Cw = _proj2(cc, dt)
    ctxc = jnp.dot(ctx[...], mc[...])
    acc = _pair_sum(th, nw, bn, Ab, qc, ac, Cw, ctxc, cc, db0[...],
                    d1T[...], d1g[...], d1b[...], md[...], c0g[...],
                    c0b[...])
    o = base + jnp.dot(acc, mc1[...])
    o = jax.nn.relu(_ln(o, ng[...], nb2[...]))
    o = _ln(jnp.dot(o, mlin[...]), lg[...], lb[...])
    out[...] = jax.nn.relu(o + a)


def _att_narrow(th, agts, actr, ctx, cctr, w, bn):
    n = agts.shape[0]
    nw = ctx.shape[0]
    grid = n // bn
    wnames = ("wqT", "qg", "qb", "agtwT", "dw0T", "db0", "d1T", "d1g", "d1b",
              "md", "c0g", "c0b", "mq", "mc", "mc1", "ng", "nb2", "mlin",
              "lg", "lb")
    wspecs = [_full(w[k].shape) for k in wnames]
    return pl.pallas_call(
        functools.partial(_att_narrow_body, th, nw),
        grid=(grid,),
        in_specs=[
            pl.BlockSpec((bn, D), lambda i: (i, 0)),
            pl.BlockSpec((bn, 2), lambda i: (i, 0)),
            _full((nw, D)),
            _full((nw, 2)),
        ] + wspecs,
        out_specs=pl.BlockSpec((bn, D), lambda i: (i, 0)),
        out_shape=jax.ShapeDtypeStruct((n, D), jnp.float32),
    )(agts, actr, ctx, cctr, *[w[k] for k in wnames])


def _att_wide_body(th, ncb, agts, actr, ctx, cctr, wqT, qg, qb, agtwT, dw0T,
                   db0, d1T, d1g, d1b, md, c0g, c0b, mq, mc, mc1, ng, nb2,
                   mlin, lg, lb, out, acc):
    k = pl.program_id(0)

    @pl.when(k == 0)
    def _():
        acc[...] = jnp.zeros((NA, D), jnp.float32)

    a = agts[...]
    ac = actr[...]
    cc = cctr[...]
    dt = dw0T[...]
    q = jax.nn.relu(_ln(jnp.dot(a, wqT[...]), qg[...], qb[...]))
    qc = jnp.dot(q, mq[...])
    Ab = _proj2(ac, dt)
    Cw = _proj2(cc, dt)
    ctxc = jnp.dot(ctx[...], mc[...])
    acc[...] += _pair_sum(th, CB, NA, Ab, qc, ac, Cw, ctxc, cc, db0[...],
                          d1T[...], d1g[...], d1b[...], md[...], c0g[...],
                          c0b[...])

    @pl.when(k == ncb - 1)
    def _():
        base = jnp.dot(a, agtwT[...])
        o = base + jnp.dot(acc[...], mc1[...])
        o = jax.nn.relu(_ln(o, ng[...], nb2[...]))
        o = _ln(jnp.dot(o, mlin[...]), lg[...], lb[...])
        out[...] = jax.nn.relu(o + a)


def _att_wide(th, agts, actr, ctx, cctr, w):
    nc = ctx.shape[0]
    ncb = nc // CB
    wnames = ("wqT", "qg", "qb", "agtwT", "dw0T", "db0", "d1T", "d1g", "d1b",
              "md", "c0g", "c0b", "mq", "mc", "mc1", "ng", "nb2", "mlin",
              "lg", "lb")
    wspecs = [_full(w[k].shape) for k in wnames]
    return pl.pallas_call(
        functools.partial(_att_wide_body, th, ncb),
        grid=(ncb,),
        in_specs=[
            _full((NA, D)),
            _full((NA, 2)),
            pl.BlockSpec((CB, D), lambda i: (i, 0)),
            pl.BlockSpec((CB, 2), lambda i: (i, 0)),
        ] + wspecs,
        out_specs=_full((NA, D)),
        out_shape=jax.ShapeDtypeStruct((NA, D), jnp.float32),
        scratch_shapes=[pltpu.VMEM((NA, D), jnp.float32)],
    )(agts, actr, ctx, cctr, *[w[k] for k in wnames])


def _m2m_pre_body(feat, wT, out):
    out[...] = jnp.dot(feat[...], wT[0])[None]


def _m2m_pre(feat, wT):
    n = feat.shape[0]
    nb = n // BN
    return pl.pallas_call(
        _m2m_pre_body,
        grid=(5, nb),
        in_specs=[
            pl.BlockSpec((BN, D), lambda e, i: (i, 0)),
            pl.BlockSpec((1, D, D), lambda e, i: (e, 0, 0)),
        ],
        out_specs=pl.BlockSpec((1, BN, D), lambda e, i: (e, i, 0)),
        out_shape=jax.ShapeDtypeStruct((5, n, D), jnp.float32),
    )(feat, wT)


def _edge_agg_body(npad, nbatch, rows_per_sub, G, u, v, z, out, idx_u, idx_v,
                   rows, shared, gsem, ssem0, ssem1):
    cid = lax.axis_index("c")
    sid = lax.axis_index("s")
    wid = cid * 16 + sid
    rbase = pl.multiple_of(sid * rows_per_sub, 8)
    pltpu.sync_copy(z, shared.at[pl.ds(rbase, rows_per_sub)])
    plsc.subcore_barrier()
    ebase = wid * (nbatch * EB)
    ssems = (ssem0, ssem1)

    def body2(t, carry):
        for s in (0, 1):
            b = 2 * t + s
            off = pl.multiple_of(ebase + b * EB, EB)

            @pl.when(t > 0)
            def _():
                pltpu.make_async_copy(rows.at[s], shared.at[idx_u.at[s]],
                                      ssems[s]).wait()

            pltpu.sync_copy(u.at[pl.ds(off, EB)], idx_u.at[s])
            pltpu.sync_copy(v.at[pl.ds(off, EB)], idx_v.at[s])
            pltpu.async_copy(G.at[idx_v.at[s]], rows.at[s], gsem).wait()
            pltpu.async_copy(rows.at[s], shared.at[idx_u.at[s]], ssems[s],
                             add=True)
        return carry

    lax.fori_loop(0, nbatch // 2, body2, 0)
    for s in (0, 1):
        pltpu.make_async_copy(rows.at[s], shared.at[idx_u.at[s]],
                              ssems[s]).wait()
    plsc.subcore_barrier()
    obase = pl.multiple_of(cid * npad + rbase, 8)
    pltpu.sync_copy(shared.at[pl.ds(rbase, rows_per_sub)],
                    out.at[pl.ds(obase, rows_per_sub)])


def _edge_agg(G, u, v, z, npad, nbatch, rows_per_sub):
    mesh = plsc.VectorSubcoreMesh(core_axis_name="c", subcore_axis_name="s")
    return pl.kernel(
        functools.partial(_edge_agg_body, npad, nbatch, rows_per_sub),
        out_type=jax.ShapeDtypeStruct((2 * npad, D), jnp.float32),
        mesh=mesh,
        scratch_types=[
            pltpu.VMEM((2, EB), jnp.int32),
            pltpu.VMEM((2, EB), jnp.int32),
            pltpu.VMEM((2, EB, D), jnp.float32),
            pltpu.VMEM_SHARED((npad, D), jnp.float32),
            pltpu.SemaphoreType.DMA,
            pltpu.SemaphoreType.DMA,
            pltpu.SemaphoreType.DMA,
        ],
    )(G, u, v, z)


def _m2m_post_body(base, agg, feat, ng, nb2, c2T, c2g, c2b, out):
    temp = base[0] + agg[0] + agg[1]
    x = jax.nn.relu(_ln(temp, ng[...], nb2[...]))
    y = _ln(jnp.dot(x, c2T[...]), c2g[...], c2b[...])
    out[...] = jax.nn.relu(y + feat[...])


def _m2m_post(base, agg, feat, ng, nb2, c2T, c2g, c2b):
    n = feat.shape[0]
    nb = n // BN
    return pl.pallas_call(
        _m2m_post_body,
        grid=(nb,),
        in_specs=[
            pl.BlockSpec((1, BN, D), lambda i: (0, i, 0)),
            pl.BlockSpec((2, BN, D), lambda i: (0, i, 0)),
            pl.BlockSpec((BN, D), lambda i: (i, 0)),
            _full((1, D)), _full((1, D)), _full((D, D)), _full((1, D)),
            _full((1, D)),
        ],
        out_specs=pl.BlockSpec((BN, D), lambda i: (i, 0)),
        out_shape=jax.ShapeDtypeStruct((n, D), jnp.float32),
    )(base, agg, feat, ng, nb2, c2T, c2g, c2b)


def _meta_body(x, wT, g, b, out):
    out[...] = jax.nn.relu(_ln(jnp.dot(x[...], wT[...]), g[...], b[...]))


def _meta_lin(x, wT, g, b):
    n = x.shape[0]
    kdim = x.shape[1]
    nb = n // BN
    return pl.pallas_call(
        _meta_body,
        grid=(nb,),
        in_specs=[pl.BlockSpec((BN, kdim), lambda i: (i, 0)),
                  _full((kdim, D)), _full((1, D)), _full((1, D))],
        out_specs=pl.BlockSpec((BN, D), lambda i: (i, 0)),
        out_shape=jax.ShapeDtypeStruct((n, D), jnp.float32),
    )(x, wT, g, b)


def _row(x):
    return x.reshape(1, -1)


def _prep_att(p):
    c0 = p["ctx0"]["w"]
    return {
        "wqT": p["query"]["w"].T, "qg": _row(p["query"]["g"]),
        "qb": _row(p["query"]["b"]), "agtwT": p["agt_w"].T,
        "dw0T": p["dist_w0"].T, "db0": _row(p["dist_b0"]),
        "d1T": p["dist1"]["w"].T, "d1g": _row(p["dist1"]["g"]),
        "d1b": _row(p["dist1"]["b"]),
        "md": c0[:, :D].T, "mq": c0[:, D:2 * D].T, "mc": c0[:, 2 * D:].T,
        "c0g": _row(p["ctx0"]["g"]), "c0b": _row(p["ctx0"]["b"]),
        "mc1": p["ctx1_w"].T,
        "ng": _row(p["norm_g"]), "nb2": _row(p["norm_b"]),
        "mlin": p["lin"]["w"].T, "lg": _row(p["lin"]["g"]),
        "lb": _row(p["lin"]["b"]),
    }


def kernel(actors, nodes, actor_ctrs, graph_ctrs, graph_turn, graph_control,
           graph_intersect, pre_u, pre_v, suc_u, suc_v, left_u, left_v,
           right_u, right_v, params):
    n = nodes.shape[0]
    npad = ((n + BN - 1) // BN) * BN
    padn = npad - n

    nodes_p = jnp.pad(nodes, ((0, padn), (0, 0)))
    turn_p = jnp.pad(graph_turn, ((0, padn), (0, 0)))
    ctrl_p = jnp.pad(graph_control, (0, padn))[:, None]
    inter_p = jnp.pad(graph_intersect, (0, padn))[:, None]
    gctr_p = jnp.pad(graph_ctrs, ((0, padn), (0, 0)), constant_values=1e9)

    xcat = jnp.concatenate([nodes_p, turn_p, ctrl_p, inter_p], axis=1)
    kdim = 256
    xcat = jnp.pad(xcat, ((0, 0), (0, kdim - xcat.shape[1])))
    mw = params["a2m_meta"]
    metaT = jnp.pad(mw["w"].T, ((0, kdim - mw["w"].shape[1]), (0, 0)))
    feat = _meta_lin(xcat, metaT, _row(mw["g"]), _row(mw["b"]))
    for i in range(2):
        feat = _att_narrow(7.0, feat, gctr_p, actors, actor_ctrs,
                           _prep_att(params["a2m_att"][i]), BA)

    e_real = 2 * pre_u.shape[0] + 2 * left_u.shape[0]
    gran = 2 * EB
    per_w = ((e_real + NWORK * gran - 1) // (NWORK * gran)) * gran
    e_pad = NWORK * per_w
    nbatch = per_w // EB
    u_all = jnp.concatenate([pre_u, suc_u, left_u, right_u])
    v_off = jnp.concatenate([pre_v + npad, suc_v + 2 * npad,
                             left_v + 3 * npad, right_v + 4 * npad])
    u_all = jnp.pad(u_all, (0, e_pad - e_real), constant_values=npad - 1)
    v_off = jnp.pad(v_off, (0, e_pad - e_real))
    rows_per_sub = npad // 16
    zrows = jnp.zeros((rows_per_sub, D), jnp.float32)

    m = params["m2m"]
    res = feat
    for i in range(4):
        wT = jnp.stack([m["ctr"][i].T, m["pre"][i].T, m["suc"][i].T,
                        m["left"][i].T, m["right"][i].T])
        G = _m2m_pre(feat, wT)
        agg = _edge_agg(G.reshape(5 * npad, D), u_all, v_off, zrows,
                        npad, nbatch, rows_per_sub)
        feat = _m2m_post(G[0:1], agg.reshape(2, npad, D), res,
                         _row(m["norm_g"][i]), _row(m["norm_b"][i]),
                         m["ctr2_w"][i].T, _row(m["ctr2_g"][i]),
                         _row(m["ctr2_b"][i]))
        res = feat

    acts = actors
    for i in range(2):
        acts = _att_wide(6.0, acts, actor_ctrs, feat, gctr_p,
                         _prep_att(params["m2a_att"][i]))

    for i in range(2):
        acts = _att_narrow(100.0, acts, actor_ctrs, acts, actor_ctrs,
                           _prep_att(params["a2a_att"][i]), NA)

    return acts


# speedup vs baseline: 8.0466x; 1.0006x over previous
import functools

import jax
import jax.numpy as jnp
from jax import lax
from jax.experimental import pallas as pl
from jax.experimental.pallas import tpu as pltpu
from jax.experimental.pallas import tpu_sc as plsc

D = 128
NA = 64
BN = 256
BA = 128
CB = 128
EB = 128
IC = 8
NWORK = 32


def _ln(x, g, b):
    m = jnp.mean(x, axis=-1, keepdims=True)
    d = x - m
    v = jnp.mean(d * d, axis=-1, keepdims=True)
    return d * lax.rsqrt(v + 1e-5) * g + b


def _full(shape):
    n = len(shape)
    return pl.BlockSpec(shape, lambda *_: (0,) * n)


def _bdot(x, wT):
    return jnp.dot(x.astype(jnp.bfloat16), wT.astype(jnp.bfloat16),
                   preferred_element_type=jnp.float32)


def _pair_sum(th, nw, bn, Ab, qc, ac, Cw, ctxc, cc, b0, d1T, d1g, d1b, md,
              c0g, c0b):
    P = nw * bn
    d0 = jax.nn.relu((Ab[None, :, :] - Cw[:, None, :]
                      + b0[None, :, :]).reshape(P, D))
    d1 = jax.nn.relu(_ln(_bdot(d0, d1T), d1g, d1b))
    qcp = jnp.broadcast_to(qc[None, :, :], (nw, bn, D)).reshape(P, D)
    cxp = jnp.broadcast_to(ctxc[:, None, :], (nw, bn, D)).reshape(P, D)
    h = jax.nn.relu(_ln(_bdot(d1, md) + qcp + cxp, c0g, c0b))
    dxy = (ac[None, :, :] - cc[:, None, :]).reshape(P, 2)
    dsq = dxy[:, 0:1] * dxy[:, 0:1] + dxy[:, 1:2] * dxy[:, 1:2]
    mh = jnp.where(jnp.sqrt(dsq) <= th, h, 0.0)
    m = nw
    while m > 1:
        m //= 2
        mh = mh[: m * bn] + mh[m * bn:]
    return mh


def _proj2(xy, dt):
    return xy[:, 0:1] * dt[0:1, :] + xy[:, 1:2] * dt[1:2, :]


def _att_narrow_body(th, nw, agts, actr, ctx, cctr, wqT, qg, qb, agtwT, dw0T,
                     db0, d1T, d1g, d1b, md, c0g, c0b, mq, mc, mc1, ng, nb2,
                     mlin, lg, lb, out):
    a = agts[...]
    bn = a.shape[0]
    q = jax.nn.relu(_ln(jnp.dot(a, wqT[...]), qg[...], qb[...]))
    qc = jnp.dot(q, mq[...])
    base = jnp.dot(a, agtwT[...])
    ac = actr[...]
    cc = cctr[...]
    dt = dw0T[...]
    Ab = _proj2(ac, dt)
    Cw = _proj2(cc, dt)
    ctxc = jnp.dot(ctx[...], mc[...])
    acc = _pair_sum(th, nw, bn, Ab, qc, ac, Cw, ctxc, cc, db0[...],
                    d1T[...], d1g[...], d1b[...], md[...], c0g[...],
                    c0b[...])
    o = base + jnp.dot(acc, mc1[...])
    o = jax.nn.relu(_ln(o, ng[...], nb2[...]))
    o = _ln(jnp.dot(o, mlin[...]), lg[...], lb[...])
    out[...] = jax.nn.relu(o + a)


def _att_narrow(th, agts, actr, ctx, cctr, w, bn):
    n = agts.shape[0]
    nw = ctx.shape[0]
    grid = n // bn
    wnames = ("wqT", "qg", "qb", "agtwT", "dw0T", "db0", "d1T", "d1g", "d1b",
              "md", "c0g", "c0b", "mq", "mc", "mc1", "ng", "nb2", "mlin",
              "lg", "lb")
    wspecs = [_full(w[k].shape) for k in wnames]
    return pl.pallas_call(
        functools.partial(_att_narrow_body, th, nw),
        grid=(grid,),
        in_specs=[
            pl.BlockSpec((bn, D), lambda i: (i, 0)),
            pl.BlockSpec((bn, 2), lambda i: (i, 0)),
            _full((nw, D)),
            _full((nw, 2)),
        ] + wspecs,
        out_specs=pl.BlockSpec((bn, D), lambda i: (i, 0)),
        out_shape=jax.ShapeDtypeStruct((n, D), jnp.float32),
    )(agts, actr, ctx, cctr, *[w[k] for k in wnames])


def _att_wide_body(th, ncb, agts, actr, ctx, cctr, wqT, qg, qb, agtwT, dw0T,
                   db0, d1T, d1g, d1b, md, c0g, c0b, mq, mc, mc1, ng, nb2,
                   mlin, lg, lb, out, acc):
    k = pl.program_id(0)

    @pl.when(k == 0)
    def _():
        acc[...] = jnp.zeros((NA, D), jnp.float32)

    a = agts[...]
    ac = actr[...]
    cc = cctr[...]
    dt = dw0T[...]
    q = jax.nn.relu(_ln(jnp.dot(a, wqT[...]), qg[...], qb[...]))
    qc = jnp.dot(q, mq[...])
    Ab = _proj2(ac, dt)
    Cw = _proj2(cc, dt)
    ctxc = jnp.dot(ctx[...], mc[...])
    acc[...] += _pair_sum(th, CB, NA, Ab, qc, ac, Cw, ctxc, cc, db0[...],
                          d1T[...], d1g[...], d1b[...], md[...], c0g[...],
                          c0b[...])

    @pl.when(k == ncb - 1)
    def _():
        base = jnp.dot(a, agtwT[...])
        o = base + jnp.dot(acc[...], mc1[...])
        o = jax.nn.relu(_ln(o, ng[...], nb2[...]))
        o = _ln(jnp.dot(o, mlin[...]), lg[...], lb[...])
        out[...] = jax.nn.relu(o + a)


def _att_wide(th, agts, actr, ctx, cctr, w):
    nc = ctx.shape[0]
    ncb = nc // CB
    wnames = ("wqT", "qg", "qb", "agtwT", "dw0T", "db0", "d1T", "d1g", "d1b",
              "md", "c0g", "c0b", "mq", "mc", "mc1", "ng", "nb2", "mlin",
              "lg", "lb")
    wspecs = [_full(w[k].shape) for k in wnames]
    return pl.pallas_call(
        functools.partial(_att_wide_body, th, ncb),
        grid=(ncb,),
        in_specs=[
            _full((NA, D)),
            _full((NA, 2)),
            pl.BlockSpec((CB, D), lambda i: (i, 0)),
            pl.BlockSpec((CB, 2), lambda i: (i, 0)),
        ] + wspecs,
        out_specs=_full((NA, D)),
        out_shape=jax.ShapeDtypeStruct((NA, D), jnp.float32),
        scratch_shapes=[pltpu.VMEM((NA, D), jnp.float32)],
    )(agts, actr, ctx, cctr, *[w[k] for k in wnames])


def _m2m_pre_body(feat, wT, out):
    out[...] = jnp.dot(feat[...], wT[0])[None]


def _m2m_pre(feat, wT):
    n = feat.shape[0]
    nb = n // BN
    return pl.pallas_call(
        _m2m_pre_body,
        grid=(5, nb),
        in_specs=[
            pl.BlockSpec((BN, D), lambda e, i: (i, 0)),
            pl.BlockSpec((1, D, D), lambda e, i: (e, 0, 0)),
        ],
        out_specs=pl.BlockSpec((1, BN, D), lambda e, i: (e, i, 0)),
        out_shape=jax.ShapeDtypeStruct((5, n, D), jnp.float32),
    )(feat, wT)


def _edge_agg_body(npad, nbatch, rows_per_sub, G, u, v, z, out, idx_u, idx_v,
                   rows, shared, gsem, ssem0, ssem1):
    cid = lax.axis_index("c")
    sid = lax.axis_index("s")
    wid = cid * 16 + sid
    rbase = pl.multiple_of(sid * rows_per_sub, 8)
    pltpu.sync_copy(z, shared.at[pl.ds(rbase, rows_per_sub)])
    plsc.subcore_barrier()
    ssems = (ssem0, ssem1)

    def chunk(t, carry):
        @pl.when(t > 0)
        def _():
            for s in (0, 1):
                pltpu.make_async_copy(rows.at[s],
                                      shared.at[idx_u.at[IC - 2 + s]],
                                      ssems[s]).wait()

        pltpu.sync_copy(u.at[wid, pl.ds(t * IC, IC)], idx_u)
        pltpu.sync_copy(v.at[wid, pl.ds(t * IC, IC)], idx_v)
        for j in range(IC):
            s = j % 2
            if j >= 2:
                pltpu.make_async_copy(rows.at[s],
                                      shared.at[idx_u.at[j - 2]],
                                      ssems[s]).wait()
            pltpu.async_copy(G.at[idx_v.at[j]], rows.at[s], gsem).wait()
            pltpu.async_copy(rows.at[s], shared.at[idx_u.at[j]], ssems[s],
                             add=True)
        return carry

    lax.fori_loop(0, nbatch // IC, chunk, 0)
    for s in (0, 1):
        pltpu.make_async_copy(rows.at[s], shared.at[idx_u.at[IC - 2 + s]],
                              ssems[s]).wait()
    plsc.subcore_barrier()
    obase = pl.multiple_of(cid * npad + rbase, 8)
    pltpu.sync_copy(shared.at[pl.ds(rbase, rows_per_sub)],
                    out.at[pl.ds(obase, rows_per_sub)])


def _edge_agg(G, u, v, z, npad, nbatch, rows_per_sub):
    mesh = plsc.VectorSubcoreMesh(core_axis_name="c", subcore_axis_name="s")
    return pl.kernel(
        functools.partial(_edge_agg_body, npad, nbatch, rows_per_sub),
        out_type=jax.ShapeDtypeStruct((2 * npad, D), jnp.float32),
        mesh=mesh,
        scratch_types=[
            pltpu.VMEM((IC, EB), jnp.int32),
            pltpu.VMEM((IC, EB), jnp.int32),
            pltpu.VMEM((2, EB, D), jnp.float32),
            pltpu.VMEM_SHARED((npad, D), jnp.float32),
            pltpu.SemaphoreType.DMA,
            pltpu.SemaphoreType.DMA,
            pltpu.SemaphoreType.DMA,
        ],
    )(G, u, v, z)


def _m2m_post_body(base, agg, feat, ng, nb2, c2T, c2g, c2b, out):
    temp = base[0] + agg[0] + agg[1]
    x = jax.nn.relu(_ln(temp, ng[...], nb2[...]))
    y = _ln(jnp.dot(x, c2T[...]), c2g[...], c2b[...])
    out[...] = jax.nn.relu(y + feat[...])


def _m2m_post(base, agg, feat, ng, nb2, c2T, c2g, c2b):
    n = feat.shape[0]
    nb = n // BN
    return pl.pallas_call(
        _m2m_post_body,
        grid=(nb,),
        in_specs=[
            pl.BlockSpec((1, BN, D), lambda i: (0, i, 0)),
            pl.BlockSpec((2, BN, D), lambda i: (0, i, 0)),
            pl.BlockSpec((BN, D), lambda i: (i, 0)),
            _full((1, D)), _full((1, D)), _full((D, D)), _full((1, D)),
            _full((1, D)),
        ],
        out_specs=pl.BlockSpec((BN, D), lambda i: (i, 0)),
        out_shape=jax.ShapeDtypeStruct((n, D), jnp.float32),
    )(base, agg, feat, ng, nb2, c2T, c2g, c2b)


def _meta_body(x, wT, g, b, out):
    out[...] = jax.nn.relu(_ln(jnp.dot(x[...], wT[...]), g[...], b[...]))


def _meta_lin(x, wT, g, b):
    n = x.shape[0]
    kdim = x.shape[1]
    nb = n // BN
    return pl.pallas_call(
        _meta_body,
        grid=(nb,),
        in_specs=[pl.BlockSpec((BN, kdim), lambda i: (i, 0)),
                  _full((kdim, D)), _full((1, D)), _full((1, D))],
        out_specs=pl.BlockSpec((BN, D), lambda i: (i, 0)),
        out_shape=jax.ShapeDtypeStruct((n, D), jnp.float32),
    )(x, wT, g, b)


def _row(x):
    return x.reshape(1, -1)


def _prep_att(p):
    c0 = p["ctx0"]["w"]
    return {
        "wqT": p["query"]["w"].T, "qg": _row(p["query"]["g"]),
        "qb": _row(p["query"]["b"]), "agtwT": p["agt_w"].T,
        "dw0T": p["dist_w0"].T, "db0": _row(p["dist_b0"]),
        "d1T": p["dist1"]["w"].T, "d1g": _row(p["dist1"]["g"]),
        "d1b": _row(p["dist1"]["b"]),
        "md": c0[:, :D].T, "mq": c0[:, D:2 * D].T, "mc": c0[:, 2 * D:].T,
        "c0g": _row(p["ctx0"]["g"]), "c0b": _row(p["ctx0"]["b"]),
        "mc1": p["ctx1_w"].T,
        "ng": _row(p["norm_g"]), "nb2": _row(p["norm_b"]),
        "mlin": p["lin"]["w"].T, "lg": _row(p["lin"]["g"]),
        "lb": _row(p["lin"]["b"]),
    }


def kernel(actors, nodes, actor_ctrs, graph_ctrs, graph_turn, graph_control,
           graph_intersect, pre_u, pre_v, suc_u, suc_v, left_u, left_v,
           right_u, right_v, params):
    n = nodes.shape[0]
    npad = ((n + BN - 1) // BN) * BN
    padn = npad - n

    nodes_p = jnp.pad(nodes, ((0, padn), (0, 0)))
    turn_p = jnp.pad(graph_turn, ((0, padn), (0, 0)))
    ctrl_p = jnp.pad(graph_control, (0, padn))[:, None]
    inter_p = jnp.pad(graph_intersect, (0, padn))[:, None]
    gctr_p = jnp.pad(graph_ctrs, ((0, padn), (0, 0)), constant_values=1e9)

    xcat = jnp.concatenate([nodes_p, turn_p, ctrl_p, inter_p], axis=1)
    kdim = 256
    xcat = jnp.pad(xcat, ((0, 0), (0, kdim - xcat.shape[1])))
    mw = params["a2m_meta"]
    metaT = jnp.pad(mw["w"].T, ((0, kdim - mw["w"].shape[1]), (0, 0)))
    feat = _meta_lin(xcat, metaT, _row(mw["g"]), _row(mw["b"]))
    for i in range(2):
        feat = _att_narrow(7.0, feat, gctr_p, actors, actor_ctrs,
                           _prep_att(params["a2m_att"][i]), BA)

    e_real = 2 * pre_u.shape[0] + 2 * left_u.shape[0]
    gran = IC * EB
    per_w = ((e_real + NWORK * gran - 1) // (NWORK * gran)) * gran
    e_pad = NWORK * per_w
    nbatch = per_w // EB
    u_all = jnp.concatenate([pre_u, suc_u, left_u, right_u])
    v_off = jnp.concatenate([pre_v + npad, suc_v + 2 * npad,
                             left_v + 3 * npad, right_v + 4 * npad])
    u_all = jnp.pad(u_all, (0, e_pad - e_real),
                    constant_values=npad - 1).reshape(NWORK, nbatch, EB)
    v_off = jnp.pad(v_off, (0, e_pad - e_real)).reshape(NWORK, nbatch, EB)
    rows_per_sub = npad // 16
    zrows = jnp.zeros((rows_per_sub, D), jnp.float32)

    m = params["m2m"]
    res = feat
    for i in range(4):
        wT = jnp.stack([m["ctr"][i].T, m["pre"][i].T, m["suc"][i].T,
                        m["left"][i].T, m["right"][i].T])
        G = _m2m_pre(feat, wT)
        agg = _edge_agg(G.reshape(5 * npad, D), u_all, v_off, zrows,
                        npad, nbatch, rows_per_sub)
        feat = _m2m_post(G[0:1], agg.reshape(2, npad, D), res,
                         _row(m["norm_g"][i]), _row(m["norm_b"][i]),
                         m["ctr2_w"][i].T, _row(m["ctr2_g"][i]),
                         _row(m["ctr2_b"][i]))
        res = feat

    acts = actors
    for i in range(2):
        acts = _att_wide(6.0, acts, actor_ctrs, feat, gctr_p,
                         _prep_att(params["m2a_att"][i]))

    for i in range(2):
        acts = _att_narrow(100.0, acts, actor_ctrs, acts, actor_ctrs,
                           _prep_att(params["a2a_att"][i]), NA)

    return acts
